# baseline (speedup 1.0000x reference)
import jax
import jax.numpy as jnp
from jax import lax
from jax.experimental import pallas as pl
from jax.experimental.pallas import tpu as pltpu
from jax.experimental.pallas import tpu_sc as plsc

V_DIM = 100000
D_DIM = 64
BATCH = 4096
HIST_LEN = 50

NC = 2
NS = 16
NW = NC * NS
LANES = 16

D_PER_W = D_DIM // NW
N_IDX = BATCH * HIST_LEN
SG_PER_W = N_IDX // NW
SG_SHOT = SG_PER_W // 2
H_VECS = BATCH // LANES
H_CACHED = 32


def _softplus_vec(v):
    a = jnp.abs(v)
    u = jnp.exp(-a)
    t = u / (2.0 + u)
    t2 = t * t
    s = 1.0 + t2 * (1.0 / 3.0 + t2 * (1.0 / 5.0 + t2 * (1.0 / 7.0 + t2 * (1.0 / 9.0))))
    return jnp.maximum(v, 0.0) + 2.0 * t * s


def _sc_body(xt_hbm, mu_t_hbm, sg_t_hbm, mu_out_hbm, sg_out_hbm,
             row_v, idx_v, out_v, sgi_v, sg_v, xt_sp,
             sem_row, sem_idx, sem_out, sem_sgi, sem_sg, sem_sgo, sem_xs):
    c = lax.axis_index("c")
    s = lax.axis_index("s")
    wid = s * NC + c

    pltpu.make_async_copy(mu_t_hbm.at[wid * D_PER_W], row_v, sem_row).start()
    for j in range(H_CACHED // NS):
        h = s + j * NS
        pltpu.make_async_copy(
            xt_hbm.at[pl.ds(h * BATCH, BATCH)],
            xt_sp.at[pl.ds(h * BATCH, BATCH)], sem_xs).start()

    sg_base = wid * SG_PER_W
    pltpu.make_async_copy(
        xt_hbm.at[pl.ds(sg_base, SG_SHOT)], sgi_v, sem_sgi).start()
    pltpu.make_async_copy(
        xt_hbm.at[pl.ds(sg_base, SG_SHOT)], sgi_v, sem_sgi).wait()
    pltpu.make_async_copy(sg_t_hbm.at[sgi_v], sg_v, sem_sg).start()

    for j in range(H_CACHED // NS):
        h = s + j * NS
        pltpu.make_async_copy(
            xt_hbm.at[pl.ds(h * BATCH, BATCH)],
            xt_sp.at[pl.ds(h * BATCH, BATCH)], sem_xs).wait()
    plsc.subcore_barrier()

    def sigma_finish(shot):
        base = sg_base + shot * SG_SHOT
        pltpu.make_async_copy(sg_t_hbm.at[sgi_v], sg_v, sem_sg).wait()

        def sp_step(i):
            off = i * LANES
            sg_v[pl.ds(off, LANES)] = _softplus_vec(sg_v[pl.ds(off, LANES)])

        plsc.parallel_loop(0, SG_SHOT // LANES, unroll=4)(sp_step)
        pltpu.make_async_copy(
            sg_v, sg_out_hbm.at[pl.ds(base, SG_SHOT)], sem_sgo).start()
        if shot == 0:
            pltpu.make_async_copy(
                xt_hbm.at[pl.ds(sg_base + SG_SHOT, SG_SHOT)],
                sgi_v, sem_sgi).start()
            pltpu.make_async_copy(
                xt_hbm.at[pl.ds(sg_base + SG_SHOT, SG_SHOT)],
                sgi_v, sem_sgi).wait()
            pltpu.make_async_copy(sg_t_hbm.at[sgi_v], sg_v, sem_sg).start()

    def idx_start(h, p):
        cp_sp = pltpu.make_async_copy(
            xt_sp.at[pl.ds(jnp.minimum(h, H_CACHED - 1) * BATCH, BATCH)],
            idx_v[p], sem_idx[p])
        cp_hbm = pltpu.make_async_copy(
            xt_hbm.at[pl.ds(h * BATCH, BATCH)], idx_v[p], sem_idx[p])

        @pl.when(h < H_CACHED)
        def _from_spmem():
            cp_sp.start()

        @pl.when(h >= H_CACHED)
        def _from_hbm():
            cp_hbm.start()

    for plane in range(D_PER_W):
        d = wid * D_PER_W + plane
        if plane > 0:
            pltpu.make_async_copy(mu_t_hbm.at[d], row_v, sem_row).start()

        idx_start(0, 0)
        pltpu.make_async_copy(mu_t_hbm.at[d], row_v, sem_row).wait()

        def h_pair(i, carry):
            for par in range(2):
                h = 2 * i + par
                @pl.when(h + 1 < HIST_LEN)
                def _start_next():
                    idx_start(h + 1, 1 - par)

                pltpu.make_async_copy(
                    xt_hbm.at[pl.ds(h * BATCH, BATCH)],
                    idx_v[par], sem_idx[par]).wait()

                @pl.when(i > 0)
                def _drain_prev():
                    pltpu.make_async_copy(
                        out_v[par],
                        mu_out_hbm.at[jnp.maximum(h - 2, 0), d],
                        sem_out[par]).wait()

                def g_step(j):
                    off = j * LANES
                    iv = idx_v[par][pl.ds(off, LANES)]
                    out_v[par][pl.ds(off, LANES)] = plsc.load_gather(row_v, [iv])

                plsc.parallel_loop(0, H_VECS, unroll=16)(g_step)
                pltpu.make_async_copy(
                    out_v[par], mu_out_hbm.at[h, d], sem_out[par]).start()
            return carry

        lax.fori_loop(0, HIST_LEN // 2, h_pair, None)
        for par in range(2):
            pltpu.make_async_copy(
                out_v[par],
                mu_out_hbm.at[HIST_LEN - 2 + par, d],
                sem_out[par]).wait()
        sigma_finish(plane)
    for shot in range(2):
        pltpu.make_async_copy(
            sg_v, sg_out_hbm.at[pl.ds(sg_base + shot * SG_SHOT, SG_SHOT)],
            sem_sgo).wait()


@jax.jit
def _run(xt_flat, mu_tt, sg_flat):
    mesh = plsc.VectorSubcoreMesh(core_axis_name="c", subcore_axis_name="s")
    f = pl.kernel(
        _sc_body,
        out_type=[
            jax.ShapeDtypeStruct((HIST_LEN, D_DIM, BATCH), jnp.float32),
            jax.ShapeDtypeStruct((N_IDX,), jnp.float32),
        ],
        mesh=mesh,
        scratch_types=[
            pltpu.VMEM((V_DIM,), jnp.float32),
            [pltpu.VMEM((BATCH,), jnp.int32) for _ in range(2)],
            [pltpu.VMEM((BATCH,), jnp.float32) for _ in range(2)],
            pltpu.VMEM((SG_SHOT,), jnp.int32),
            pltpu.VMEM((SG_SHOT,), jnp.float32),
            pltpu.VMEM_SHARED((H_CACHED * BATCH,), jnp.int32),
            pltpu.SemaphoreType.DMA,
            [pltpu.SemaphoreType.DMA for _ in range(2)],
            [pltpu.SemaphoreType.DMA for _ in range(2)],
            pltpu.SemaphoreType.DMA,
            pltpu.SemaphoreType.DMA,
            pltpu.SemaphoreType.DMA,
            pltpu.SemaphoreType.DMA,
        ],
        compiler_params=pltpu.CompilerParams(
            use_tc_tiling_on_sc=True, disable_bounds_checks=True,
            needs_layout_passes=False),
    )
    return f(xt_flat, mu_tt, sg_flat)


def kernel(x, mu_table, sigma_table):
    xt_flat = x.T.reshape(N_IDX)
    mu_tt = mu_table.T
    sg_flat = sigma_table.reshape(V_DIM)
    mu_t, sg_t = _run(xt_flat, mu_tt, sg_flat)
    mu = jnp.transpose(mu_t, (2, 0, 1))
    sigma = jnp.transpose(sg_t.reshape(HIST_LEN, BATCH), (1, 0)).reshape(
        BATCH, HIST_LEN, 1)
    return (mu, sigma)


# speedup vs baseline: 1.1237x; 1.1237x over previous
import jax
import jax.numpy as jnp
from jax import lax
from jax.experimental import pallas as pl
from jax.experimental.pallas import tpu as pltpu
from jax.experimental.pallas import tpu_sc as plsc

V_DIM = 100000
D_DIM = 64
BATCH = 4096
HIST_LEN = 50

NC = 2
NS = 16
NW = NC * NS
LANES = 16

D_PER_W = D_DIM // NW
N_IDX = BATCH * HIST_LEN
SG_PER_W = N_IDX // NW
SG_SHOT = SG_PER_W // 4
H_VECS = BATCH // LANES
H_CACHED = 44


def _softplus_vec(v):
    a = jnp.abs(v)
    u = jnp.exp(-a)
    t = u / (2.0 + u)
    t2 = t * t
    s = 1.0 + t2 * (1.0 / 3.0 + t2 * (1.0 / 5.0 + t2 * (1.0 / 7.0 + t2 * (1.0 / 9.0))))
    return jnp.maximum(v, 0.0) + 2.0 * t * s


def _sc_body(xt_hbm, mu_t_hbm, sg_t_hbm, mu_out_hbm, sg_out_hbm,
             row_v, idx_v, out_v, sgi_v, sg_v, xt_sp,
             sem_row, sem_idx, sem_out, sem_sgi, sem_sg, sem_sgo, sem_xs):
    c = lax.axis_index("c")
    s = lax.axis_index("s")
    wid = s * NC + c

    pltpu.make_async_copy(mu_t_hbm.at[wid * D_PER_W], row_v, sem_row).start()
    N_STAGE_ROUNDS = (H_CACHED + NS - 1) // NS

    def stage_cp(j):
        h = jnp.minimum(s + j * NS, H_CACHED - 1)
        return pltpu.make_async_copy(
            xt_hbm.at[pl.ds(h * BATCH, BATCH)],
            xt_sp.at[pl.ds(h * BATCH, BATCH)], sem_xs)

    for j in range(N_STAGE_ROUNDS):
        cpj = stage_cp(j)

        @pl.when(s + j * NS < H_CACHED)
        def _stage_start():
            cpj.start()

    sg_base = wid * SG_PER_W
    pltpu.make_async_copy(
        xt_hbm.at[pl.ds(sg_base, SG_SHOT)], sgi_v, sem_sgi).start()
    pltpu.make_async_copy(
        xt_hbm.at[pl.ds(sg_base, SG_SHOT)], sgi_v, sem_sgi).wait()
    pltpu.make_async_copy(sg_t_hbm.at[sgi_v], sg_v, sem_sg).start()

    for j in range(N_STAGE_ROUNDS):
        cpj = stage_cp(j)

        @pl.when(s + j * NS < H_CACHED)
        def _stage_wait():
            cpj.wait()
    plsc.subcore_barrier()

    def sigma_round(shot):
        base = sg_base + shot * SG_SHOT
        pltpu.make_async_copy(sg_t_hbm.at[sgi_v], sg_v, sem_sg).wait()

        def sp_step(i):
            off = i * LANES
            sg_v[pl.ds(off, LANES)] = _softplus_vec(sg_v[pl.ds(off, LANES)])

        plsc.parallel_loop(0, SG_SHOT // LANES, unroll=4)(sp_step)
        pltpu.make_async_copy(
            sg_v, sg_out_hbm.at[pl.ds(base, SG_SHOT)], sem_sgo).start()
        if shot < 3:
            pltpu.make_async_copy(
                sg_v, sg_out_hbm.at[pl.ds(base, SG_SHOT)], sem_sgo).wait()
            nbase = sg_base + (shot + 1) * SG_SHOT
            pltpu.make_async_copy(
                xt_hbm.at[pl.ds(nbase, SG_SHOT)], sgi_v, sem_sgi).start()
            pltpu.make_async_copy(
                xt_hbm.at[pl.ds(nbase, SG_SHOT)], sgi_v, sem_sgi).wait()
            pltpu.make_async_copy(sg_t_hbm.at[sgi_v], sg_v, sem_sg).start()

    def idx_start(h, p):
        cp_sp = pltpu.make_async_copy(
            xt_sp.at[pl.ds(jnp.minimum(h, H_CACHED - 1) * BATCH, BATCH)],
            idx_v[p], sem_idx[p])
        cp_hbm = pltpu.make_async_copy(
            xt_hbm.at[pl.ds(h * BATCH, BATCH)], idx_v[p], sem_idx[p])

        @pl.when(h < H_CACHED)
        def _from_spmem():
            cp_sp.start()

        @pl.when(h >= H_CACHED)
        def _from_hbm():
            cp_hbm.start()

    for plane in range(D_PER_W):
        d = wid * D_PER_W + plane
        if plane > 0:
            pltpu.make_async_copy(mu_t_hbm.at[d], row_v, sem_row).start()

        idx_start(0, 0)
        pltpu.make_async_copy(mu_t_hbm.at[d], row_v, sem_row).wait()

        def h_pair(i, carry):
            for par in range(2):
                h = 2 * i + par
                @pl.when(h + 1 < HIST_LEN)
                def _start_next():
                    idx_start(h + 1, 1 - par)

                pltpu.make_async_copy(
                    xt_hbm.at[pl.ds(h * BATCH, BATCH)],
                    idx_v[par], sem_idx[par]).wait()

                @pl.when(i > 0)
                def _drain_prev():
                    pltpu.make_async_copy(
                        out_v[par],
                        mu_out_hbm.at[jnp.maximum(h - 2, 0), d],
                        sem_out[par]).wait()

                def g_step(j):
                    off = j * LANES
                    iv = idx_v[par][pl.ds(off, LANES)]
                    out_v[par][pl.ds(off, LANES)] = plsc.load_gather(row_v, [iv])

                plsc.parallel_loop(0, H_VECS, unroll=8)(g_step)
                pltpu.make_async_copy(
                    out_v[par], mu_out_hbm.at[h, d], sem_out[par]).start()

            @pl.when(i == 12)
            def _sigma_mid():
                sigma_round(2 * plane)

            return carry

        lax.fori_loop(0, HIST_LEN // 2, h_pair, None)
        for par in range(2):
            pltpu.make_async_copy(
                out_v[par],
                mu_out_hbm.at[HIST_LEN - 2 + par, d],
                sem_out[par]).wait()
        sigma_round(2 * plane + 1)
    pltpu.make_async_copy(
        sg_v, sg_out_hbm.at[pl.ds(sg_base + 3 * SG_SHOT, SG_SHOT)],
        sem_sgo).wait()


@jax.jit
def _run(xt_flat, mu_tt, sg_flat):
    mesh = plsc.VectorSubcoreMesh(core_axis_name="c", subcore_axis_name="s")
    f = pl.kernel(
        _sc_body,
        out_type=[
            jax.ShapeDtypeStruct((HIST_LEN, D_DIM, BATCH), jnp.float32),
            jax.ShapeDtypeStruct((N_IDX,), jnp.float32),
        ],
        mesh=mesh,
        scratch_types=[
            pltpu.VMEM((V_DIM,), jnp.float32),
            [pltpu.VMEM((BATCH,), jnp.int32) for _ in range(2)],
            [pltpu.VMEM((BATCH,), jnp.float32) for _ in range(2)],
            pltpu.VMEM((SG_SHOT,), jnp.int32),
            pltpu.VMEM((SG_SHOT,), jnp.float32),
            pltpu.VMEM_SHARED((H_CACHED * BATCH,), jnp.int32),
            pltpu.SemaphoreType.DMA,
            [pltpu.SemaphoreType.DMA for _ in range(2)],
            [pltpu.SemaphoreType.DMA for _ in range(2)],
            pltpu.SemaphoreType.DMA,
            pltpu.SemaphoreType.DMA,
            pltpu.SemaphoreType.DMA,
            pltpu.SemaphoreType.DMA,
        ],
        compiler_params=pltpu.CompilerParams(
            use_tc_tiling_on_sc=True, disable_bounds_checks=True,
            needs_layout_passes=False),
    )
    return f(xt_flat, mu_tt, sg_flat)


def kernel(x, mu_table, sigma_table):
    xt_flat = x.T.reshape(N_IDX)
    mu_tt = mu_table.T
    sg_flat = sigma_table.reshape(V_DIM)
    mu_t, sg_t = _run(xt_flat, mu_tt, sg_flat)
    mu = jnp.transpose(mu_t, (2, 0, 1))
    sigma = jnp.transpose(sg_t.reshape(HIST_LEN, BATCH), (1, 0)).reshape(
        BATCH, HIST_LEN, 1)
    return (mu, sigma)


# speedup vs baseline: 1.1613x; 1.0334x over previous
import jax
import jax.numpy as jnp
from jax import lax
from jax.experimental import pallas as pl
from jax.experimental.pallas import tpu as pltpu
from jax.experimental.pallas import tpu_sc as plsc

V_DIM = 100000
D_DIM = 64
BATCH = 4096
HIST_LEN = 50

NC = 2
NS = 16
NW = NC * NS
LANES = 16

D_PER_W = D_DIM // NW
N_IDX = BATCH * HIST_LEN
SG_PER_W = N_IDX // NW
SG_SHOT = SG_PER_W // 8
H_VECS = BATCH // LANES
H_CACHED = 48


def _softplus_vec(v):
    a = jnp.abs(v)
    u = jnp.exp(-a)
    t = u / (2.0 + u)
    t2 = t * t
    s = 1.0 + t2 * (1.0 / 3.0 + t2 * (1.0 / 5.0 + t2 * (1.0 / 7.0 + t2 * (1.0 / 9.0))))
    return jnp.maximum(v, 0.0) + 2.0 * t * s


def _sc_body(xt_hbm, mu_t_hbm, sg_t_hbm, mu_out_hbm, sg_out_hbm,
             row_v, idx_v, out_v, sgi_v, sg_v, xt_sp,
             sem_row, sem_idx, sem_out, sem_sgi, sem_sg, sem_sgo, sem_xs):
    c = lax.axis_index("c")
    s = lax.axis_index("s")
    wid = s * NC + c

    pltpu.make_async_copy(mu_t_hbm.at[wid * D_PER_W], row_v, sem_row).start()
    N_STAGE_ROUNDS = (H_CACHED + NS - 1) // NS

    def stage_cp(j):
        h = jnp.minimum(s + j * NS, H_CACHED - 1)
        return pltpu.make_async_copy(
            xt_hbm.at[pl.ds(h * BATCH, BATCH)],
            xt_sp.at[pl.ds(h * BATCH, BATCH)], sem_xs)

    for j in range(N_STAGE_ROUNDS):
        cpj = stage_cp(j)

        @pl.when(s + j * NS < H_CACHED)
        def _stage_start():
            cpj.start()

    sg_base = wid * SG_PER_W
    pltpu.make_async_copy(
        xt_hbm.at[pl.ds(sg_base, SG_SHOT)], sgi_v, sem_sgi).start()
    pltpu.make_async_copy(
        xt_hbm.at[pl.ds(sg_base, SG_SHOT)], sgi_v, sem_sgi).wait()
    pltpu.make_async_copy(sg_t_hbm.at[sgi_v], sg_v, sem_sg).start()

    for j in range(N_STAGE_ROUNDS):
        cpj = stage_cp(j)

        @pl.when(s + j * NS < H_CACHED)
        def _stage_wait():
            cpj.wait()
    plsc.subcore_barrier()

    def sigma_round(shot):
        base = sg_base + shot * SG_SHOT
        pltpu.make_async_copy(sg_t_hbm.at[sgi_v], sg_v, sem_sg).wait()

        def sp_step(i):
            off = i * LANES
            sg_v[pl.ds(off, LANES)] = _softplus_vec(sg_v[pl.ds(off, LANES)])

        plsc.parallel_loop(0, SG_SHOT // LANES, unroll=4)(sp_step)
        pltpu.make_async_copy(
            sg_v, sg_out_hbm.at[pl.ds(base, SG_SHOT)], sem_sgo).start()
        if shot < 7:
            pltpu.make_async_copy(
                sg_v, sg_out_hbm.at[pl.ds(base, SG_SHOT)], sem_sgo).wait()
            nbase = sg_base + (shot + 1) * SG_SHOT
            pltpu.make_async_copy(
                xt_hbm.at[pl.ds(nbase, SG_SHOT)], sgi_v, sem_sgi).start()
            pltpu.make_async_copy(
                xt_hbm.at[pl.ds(nbase, SG_SHOT)], sgi_v, sem_sgi).wait()
            pltpu.make_async_copy(sg_t_hbm.at[sgi_v], sg_v, sem_sg).start()

    def idx_start(h, p):
        cp_sp = pltpu.make_async_copy(
            xt_sp.at[pl.ds(jnp.minimum(h, H_CACHED - 1) * BATCH, BATCH)],
            idx_v[p], sem_idx[p])
        cp_hbm = pltpu.make_async_copy(
            xt_hbm.at[pl.ds(h * BATCH, BATCH)], idx_v[p], sem_idx[p])

        @pl.when(h < H_CACHED)
        def _from_spmem():
            cp_sp.start()

        @pl.when(h >= H_CACHED)
        def _from_hbm():
            cp_hbm.start()

    for plane in range(D_PER_W):
        d = wid * D_PER_W + plane
        if plane > 0:
            pltpu.make_async_copy(mu_t_hbm.at[d], row_v, sem_row).start()

        idx_start(0, 0)
        pltpu.make_async_copy(mu_t_hbm.at[d], row_v, sem_row).wait()

        def h_pair(i, carry):
            for par in range(2):
                h = 2 * i + par
                @pl.when(h + 1 < HIST_LEN)
                def _start_next():
                    idx_start(h + 1, 1 - par)

                pltpu.make_async_copy(
                    xt_hbm.at[pl.ds(h * BATCH, BATCH)],
                    idx_v[par], sem_idx[par]).wait()

                @pl.when(i > 0)
                def _drain_prev():
                    pltpu.make_async_copy(
                        out_v[par],
                        mu_out_hbm.at[jnp.maximum(h - 2, 0), d],
                        sem_out[par]).wait()

                def g_step(j):
                    off = j * LANES
                    iv = idx_v[par][pl.ds(off, LANES)]
                    out_v[par][pl.ds(off, LANES)] = plsc.load_gather(row_v, [iv])

                plsc.parallel_loop(0, H_VECS, unroll=8)(g_step)
                pltpu.make_async_copy(
                    out_v[par], mu_out_hbm.at[h, d], sem_out[par]).start()

            for k, itrig in enumerate((6, 12, 18)):
                @pl.when(i == itrig)
                def _sigma_mid():
                    sigma_round(4 * plane + k)

            return carry

        lax.fori_loop(0, HIST_LEN // 2, h_pair, None)
        for par in range(2):
            pltpu.make_async_copy(
                out_v[par],
                mu_out_hbm.at[HIST_LEN - 2 + par, d],
                sem_out[par]).wait()
        sigma_round(4 * plane + 3)
    pltpu.make_async_copy(
        sg_v, sg_out_hbm.at[pl.ds(sg_base + 7 * SG_SHOT, SG_SHOT)],
        sem_sgo).wait()


@jax.jit
def _run(xt_flat, mu_tt, sg_flat):
    mesh = plsc.VectorSubcoreMesh(core_axis_name="c", subcore_axis_name="s")
    f = pl.kernel(
        _sc_body,
        out_type=[
            jax.ShapeDtypeStruct((HIST_LEN, D_DIM, BATCH), jnp.float32),
            jax.ShapeDtypeStruct((N_IDX,), jnp.float32),
        ],
        mesh=mesh,
        scratch_types=[
            pltpu.VMEM((V_DIM,), jnp.float32),
            [pltpu.VMEM((BATCH,), jnp.int32) for _ in range(2)],
            [pltpu.VMEM((BATCH,), jnp.float32) for _ in range(2)],
            pltpu.VMEM((SG_SHOT,), jnp.int32),
            pltpu.VMEM((SG_SHOT,), jnp.float32),
            pltpu.VMEM_SHARED((H_CACHED * BATCH,), jnp.int32),
            pltpu.SemaphoreType.DMA,
            [pltpu.SemaphoreType.DMA for _ in range(2)],
            [pltpu.SemaphoreType.DMA for _ in range(2)],
            pltpu.SemaphoreType.DMA,
            pltpu.SemaphoreType.DMA,
            pltpu.SemaphoreType.DMA,
            pltpu.SemaphoreType.DMA,
        ],
        compiler_params=pltpu.CompilerParams(
            use_tc_tiling_on_sc=True, disable_bounds_checks=True,
            needs_layout_passes=False),
    )
    return f(xt_flat, mu_tt, sg_flat)


def kernel(x, mu_table, sigma_table):
    xt_flat = x.T.reshape(N_IDX)
    mu_tt = mu_table.T
    sg_flat = sigma_table.reshape(V_DIM)
    mu_t, sg_t = _run(xt_flat, mu_tt, sg_flat)
    mu = jnp.transpose(mu_t, (2, 0, 1))
    sigma = jnp.transpose(sg_t.reshape(HIST_LEN, BATCH), (1, 0)).reshape(
        BATCH, HIST_LEN, 1)
    return (mu, sigma)


# speedup vs baseline: 1.2922x; 1.1127x over previous
import jax
import jax.numpy as jnp
from jax import lax
from jax.experimental import pallas as pl
from jax.experimental.pallas import tpu as pltpu
from jax.experimental.pallas import tpu_sc as plsc

V_DIM = 100000
D_DIM = 64
BATCH = 4096
HIST_LEN = 50

NC = 2
NS = 16
NW = NC * NS
LANES = 16

D_PER_W = D_DIM // NW
N_IDX = BATCH * HIST_LEN
SG_PER_W = N_IDX // NW
SG_SHOT = SG_PER_W // 8
H_VECS = BATCH // LANES
H_CACHED = HIST_LEN


def _softplus_vec(v):
    a = jnp.abs(v)
    u = jnp.exp(-a)
    t = u / (2.0 + u)
    t2 = t * t
    s = 1.0 + t2 * (1.0 / 3.0 + t2 * (1.0 / 5.0 + t2 * (1.0 / 7.0 + t2 * (1.0 / 9.0))))
    return jnp.maximum(v, 0.0) + 2.0 * t * s


def _sc_body(xt_hbm, mu_t_hbm, sg_t_hbm, mu_out_hbm, sg_out_hbm,
             row_v, idx_v, out_v, sgi_v, sg_v, xt_sp,
             sem_row, sem_idx, sem_out, sem_sgi, sem_sg, sem_sgo, sem_xs):
    c = lax.axis_index("c")
    s = lax.axis_index("s")
    wid = s * NC + c

    pltpu.make_async_copy(mu_t_hbm.at[wid * D_PER_W], row_v, sem_row).start()
    N_STAGE_ROUNDS = (H_CACHED + NS - 1) // NS

    def stage_cp(j):
        h = jnp.minimum(s + j * NS, H_CACHED - 1)
        return pltpu.make_async_copy(
            xt_hbm.at[pl.ds(h * BATCH, BATCH)],
            xt_sp.at[pl.ds(h * BATCH, BATCH)], sem_xs)

    for j in range(N_STAGE_ROUNDS):
        cpj = stage_cp(j)

        @pl.when(s + j * NS < H_CACHED)
        def _stage_start():
            cpj.start()

    sg_base = wid * SG_PER_W

    for j in range(N_STAGE_ROUNDS):
        cpj = stage_cp(j)

        @pl.when(s + j * NS < H_CACHED)
        def _stage_wait():
            cpj.wait()
    plsc.subcore_barrier()

    pltpu.make_async_copy(
        xt_sp.at[pl.ds(sg_base, SG_SHOT)], sgi_v, sem_sgi).start()
    pltpu.make_async_copy(
        xt_sp.at[pl.ds(sg_base, SG_SHOT)], sgi_v, sem_sgi).wait()
    pltpu.make_async_copy(sg_t_hbm.at[sgi_v], sg_v, sem_sg).start()

    def sigma_round(shot):
        base = sg_base + shot * SG_SHOT
        pltpu.make_async_copy(sg_t_hbm.at[sgi_v], sg_v, sem_sg).wait()

        def sp_step(i):
            off = i * LANES
            sg_v[pl.ds(off, LANES)] = _softplus_vec(sg_v[pl.ds(off, LANES)])

        plsc.parallel_loop(0, SG_SHOT // LANES, unroll=4)(sp_step)
        pltpu.make_async_copy(
            sg_v, sg_out_hbm.at[pl.ds(base, SG_SHOT)], sem_sgo).start()
        if shot < 7:
            pltpu.make_async_copy(
                sg_v, sg_out_hbm.at[pl.ds(base, SG_SHOT)], sem_sgo).wait()
            nbase = sg_base + (shot + 1) * SG_SHOT
            pltpu.make_async_copy(
                xt_sp.at[pl.ds(nbase, SG_SHOT)], sgi_v, sem_sgi).start()
            pltpu.make_async_copy(
                xt_sp.at[pl.ds(nbase, SG_SHOT)], sgi_v, sem_sgi).wait()
            pltpu.make_async_copy(sg_t_hbm.at[sgi_v], sg_v, sem_sg).start()

    def idx_start(h, p):
        cp_sp = pltpu.make_async_copy(
            xt_sp.at[pl.ds(jnp.minimum(h, H_CACHED - 1) * BATCH, BATCH)],
            idx_v[p], sem_idx[p])
        cp_hbm = pltpu.make_async_copy(
            xt_hbm.at[pl.ds(h * BATCH, BATCH)], idx_v[p], sem_idx[p])

        @pl.when(h < H_CACHED)
        def _from_spmem():
            cp_sp.start()

        @pl.when(h >= H_CACHED)
        def _from_hbm():
            cp_hbm.start()

    for plane in range(D_PER_W):
        d = wid * D_PER_W + plane
        if plane > 0:
            pltpu.make_async_copy(mu_t_hbm.at[d], row_v, sem_row).start()

        idx_start(0, 0)
        pltpu.make_async_copy(mu_t_hbm.at[d], row_v, sem_row).wait()

        def h_pair(i, carry):
            for par in range(2):
                h = 2 * i + par
                @pl.when(h + 1 < HIST_LEN)
                def _start_next():
                    idx_start(h + 1, 1 - par)

                pltpu.make_async_copy(
                    xt_hbm.at[pl.ds(h * BATCH, BATCH)],
                    idx_v[par], sem_idx[par]).wait()

                @pl.when(i > 0)
                def _drain_prev():
                    pltpu.make_async_copy(
                        out_v[par],
                        mu_out_hbm.at[jnp.maximum(h - 2, 0), d],
                        sem_out[par]).wait()

                def g_step(j):
                    off = j * LANES
                    iv = idx_v[par][pl.ds(off, LANES)]
                    out_v[par][pl.ds(off, LANES)] = plsc.load_gather(row_v, [iv])

                plsc.parallel_loop(0, H_VECS, unroll=8)(g_step)
                pltpu.make_async_copy(
                    out_v[par], mu_out_hbm.at[h, d], sem_out[par]).start()

            for k, itrig in enumerate((6, 12, 18)):
                @pl.when(i == itrig)
                def _sigma_mid():
                    sigma_round(4 * plane + k)

            return carry

        lax.fori_loop(0, HIST_LEN // 2, h_pair, None)
        for par in range(2):
            pltpu.make_async_copy(
                out_v[par],
                mu_out_hbm.at[HIST_LEN - 2 + par, d],
                sem_out[par]).wait()
        sigma_round(4 * plane + 3)
    pltpu.make_async_copy(
        sg_v, sg_out_hbm.at[pl.ds(sg_base + 7 * SG_SHOT, SG_SHOT)],
        sem_sgo).wait()


@jax.jit
def _run(xt_flat, mu_tt, sg_flat):
    mesh = plsc.VectorSubcoreMesh(core_axis_name="c", subcore_axis_name="s")
    f = pl.kernel(
        _sc_body,
        out_type=[
            jax.ShapeDtypeStruct((HIST_LEN, D_DIM, BATCH), jnp.float32),
            jax.ShapeDtypeStruct((N_IDX,), jnp.float32),
        ],
        mesh=mesh,
        scratch_types=[
            pltpu.VMEM((V_DIM,), jnp.float32),
            [pltpu.VMEM((BATCH,), jnp.int32) for _ in range(2)],
            [pltpu.VMEM((BATCH,), jnp.float32) for _ in range(2)],
            pltpu.VMEM((SG_SHOT,), jnp.int32),
            pltpu.VMEM((SG_SHOT,), jnp.float32),
            pltpu.VMEM_SHARED((H_CACHED * BATCH,), jnp.int32),
            pltpu.SemaphoreType.DMA,
            [pltpu.SemaphoreType.DMA for _ in range(2)],
            [pltpu.SemaphoreType.DMA for _ in range(2)],
            pltpu.SemaphoreType.DMA,
            pltpu.SemaphoreType.DMA,
            pltpu.SemaphoreType.DMA,
            pltpu.SemaphoreType.DMA,
        ],
        compiler_params=pltpu.CompilerParams(
            use_tc_tiling_on_sc=True, disable_bounds_checks=True,
            needs_layout_passes=False),
    )
    return f(xt_flat, mu_tt, sg_flat)


def kernel(x, mu_table, sigma_table):
    xt_flat = x.T.reshape(N_IDX)
    mu_tt = mu_table.T
    sg_flat = sigma_table.reshape(V_DIM)
    mu_t, sg_t = _run(xt_flat, mu_tt, sg_flat)
    mu = jnp.transpose(mu_t, (2, 0, 1))
    sigma = jnp.transpose(sg_t.reshape(HIST_LEN, BATCH), (1, 0)).reshape(
        BATCH, HIST_LEN, 1)
    return (mu, sigma)
